# baseline (speedup 1.0000x reference)
import functools

import jax
import jax.numpy as jnp
from jax.experimental import pallas as pl
from jax.experimental.pallas import tpu as pltpu
from jax.experimental.pallas import tpu_sc as plsc

E = 16
C = 256
S = 2048
M = 1024
F = 4096
BUF_ROWS = E * C + C
DUMP = E * C
NEG = -1e30

W_SC = 32


def _cumsum_rows(a):
    d = 1
    n = a.shape[0]
    while d < n:
        a = a + jnp.concatenate([jnp.zeros((d, a.shape[1]), a.dtype), a[: n - d]], axis=0)
        d *= 2
    return a


def _gate_body(logits_ref, idx_ref, gsc_ref, laux_ref, cnt_ref):
    logits = logits_ref[...]
    ids = jax.lax.broadcasted_iota(jnp.int32, (S, E), 1)
    mx1 = jnp.max(logits, axis=1, keepdims=True)
    idx1 = jnp.min(jnp.where(logits == mx1, ids, E), axis=1, keepdims=True)
    m1 = ids == idx1
    masked = jnp.where(m1, NEG, logits)
    mx2 = jnp.max(masked, axis=1, keepdims=True)
    idx2 = jnp.min(jnp.where(masked == mx2, ids, E), axis=1, keepdims=True)
    m2 = ids == idx2
    m1f = m1.astype(jnp.float32)
    m2f = m2.astype(jnp.float32)

    z = jnp.exp(logits - mx1)
    gates = z / jnp.sum(z, axis=1, keepdims=True)

    cnt1 = jnp.sum(m1f, axis=0, keepdims=True)
    loc1 = _cumsum_rows(m1f) - 1.0
    loc2 = _cumsum_rows(m2f) - 1.0 + cnt1

    me = jnp.mean(gates, axis=0, keepdims=True)
    ce = cnt1 * (1.0 / S)
    laux_ref[...] = jnp.sum(me * ce, axis=1, keepdims=True) * E
    cnt_ref[...] = (cnt1 + jnp.sum(m2f, axis=0, keepdims=True)).astype(jnp.int32)

    k1m = m1f * (loc1 < C)
    k2m = m2f * (loc2 < C)
    l1 = jnp.sum(loc1 * k1m, axis=1, keepdims=True)
    l2 = jnp.sum(loc2 * k2m, axis=1, keepdims=True)
    keep1 = jnp.sum(k1m, axis=1, keepdims=True)
    keep2 = jnp.sum(k2m, axis=1, keepdims=True)
    g1s = jnp.sum(gates * k1m, axis=1, keepdims=True)
    g2s = jnp.sum(gates * k2m, axis=1, keepdims=True)
    denom = jnp.maximum(g1s + g2s, 1e-9)
    gsc_ref[...] = jnp.concatenate(
        [g1s / denom * keep1, g2s / denom * keep2], axis=1)

    pos1 = idx1 * C + jnp.minimum(l1, C - 1).astype(jnp.int32)
    pos2 = idx2 * C + jnp.minimum(l2, C - 1).astype(jnp.int32)
    dump = jnp.int32(DUMP)
    s1 = jnp.where(keep1 > 0, pos1, dump)
    s2 = jnp.where(keep2 > 0, pos2, dump)
    idx_ref[...] = jnp.concatenate([s1, s2, pos1, pos2], axis=1)


def _gate(logits):
    return pl.pallas_call(
        _gate_body,
        out_shape=[
            jax.ShapeDtypeStruct((S, 4), jnp.int32),
            jax.ShapeDtypeStruct((S, 2), jnp.float32),
            jax.ShapeDtypeStruct((1, 1), jnp.float32),
            jax.ShapeDtypeStruct((1, E), jnp.int32),
        ],
    )(logits)


FB = 2048


def _ffn_body(buf_ref, w1_ref, b1_ref, w2_ref, b2_ref, out_ref, scr_ref):
    f = pl.program_id(1)
    nf = F // FB
    xb = buf_ref[...]
    h = jnp.dot(xb, w1_ref[0].astype(jnp.bfloat16),
                preferred_element_type=jnp.float32)
    h = jnp.maximum(h + b1_ref[0, 0], 0.0).astype(jnp.bfloat16)
    acc = jnp.dot(h, w2_ref[0].astype(jnp.bfloat16),
                  preferred_element_type=jnp.float32)

    @pl.when(f == 0)
    def _():
        scr_ref[...] = acc

    @pl.when(f != 0)
    def _():
        scr_ref[...] += acc

    @pl.when(f == nf - 1)
    def _():
        out_ref[...] = (scr_ref[...] + b2_ref[0, 0]).astype(jnp.bfloat16)


def _ffn(buf, w1, b1, w2, b2):
    nf = F // FB
    return pl.pallas_call(
        _ffn_body,
        grid=(E, nf),
        in_specs=[
            pl.BlockSpec((C, M), lambda e, f: (e, 0)),
            pl.BlockSpec((1, M, FB), lambda e, f: (e, 0, f)),
            pl.BlockSpec((1, 1, FB), lambda e, f: (e, 0, f)),
            pl.BlockSpec((1, FB, M), lambda e, f: (e, f, 0)),
            pl.BlockSpec((1, 1, M), lambda e, f: (e, 0, 0)),
        ],
        out_specs=pl.BlockSpec((C, M), lambda e, f: (e, 0)),
        out_shape=jax.ShapeDtypeStruct((E * C, M), jnp.bfloat16),
        scratch_shapes=[pltpu.VMEM((C, M), jnp.float32)],
        compiler_params=pltpu.CompilerParams(
            dimension_semantics=("parallel", "arbitrary")),
    )(buf, w1, b1, w2, b2)


MW = M // 2
NW = 32
ITEMS = 2 * S
IPW = ITEMS // NW
NCH = IPW // W_SC


def _dispatch(x, scat_idx):
    mesh = plsc.VectorSubcoreMesh(core_axis_name="c", subcore_axis_name="s")

    @functools.partial(
        pl.kernel, mesh=mesh,
        out_type=jax.ShapeDtypeStruct((BUF_ROWS, MW), jnp.int32),
        scratch_types=[
            pltpu.VMEM((NCH, W_SC), jnp.int32),
            pltpu.VMEM((W_SC, MW), jnp.int32),
            pltpu.VMEM((W_SC, MW), jnp.int32),
            pltpu.SemaphoreType.DMA,
            pltpu.SemaphoreType.DMA,
        ],
    )
    def k(x_hbm, i_hbm, o_hbm, idx_v, xv0, xv1, s0, s1):
        wid = jax.lax.axis_index("s") * 2 + jax.lax.axis_index("c")
        xbase = (wid * IPW) % S
        pltpu.sync_copy(i_hbm.at[wid], idx_v)
        xvs, sems, cps = (xv0, xv1), (s0, s1), [None] * NCH
        for j in range(NCH):
            b = j % 2
            if j >= 2:
                cps[j - 2].wait()
            pltpu.sync_copy(x_hbm.at[pl.ds(xbase + j * W_SC, W_SC)], xvs[b])
            cps[j] = pltpu.async_copy(xvs[b], o_hbm.at[idx_v.at[j]], sems[b])
        cps[NCH - 2].wait()
        cps[NCH - 1].wait()

    return k(x, scat_idx)


def _gather(data, gath_idx):
    mesh = plsc.VectorSubcoreMesh(core_axis_name="c", subcore_axis_name="s")

    @functools.partial(
        pl.kernel, mesh=mesh,
        out_type=jax.ShapeDtypeStruct((ITEMS, MW), jnp.int32),
        scratch_types=[
            pltpu.VMEM((NCH, W_SC), jnp.int32),
            pltpu.VMEM((W_SC, MW), jnp.int32),
            pltpu.VMEM((W_SC, MW), jnp.int32),
            pltpu.SemaphoreType.DMA,
            pltpu.SemaphoreType.DMA,
        ],
    )
    def k(d_hbm, i_hbm, o_hbm, idx_v, rv0, rv1, s0, s1):
        wid = jax.lax.axis_index("s") * 2 + jax.lax.axis_index("c")
        base = wid * IPW
        pltpu.sync_copy(i_hbm.at[wid], idx_v)
        rvs, sems, cps = (rv0, rv1), (s0, s1), [None] * NCH
        cps[0] = pltpu.async_copy(d_hbm.at[idx_v.at[0]], rv0, s0)
        cps[1] = pltpu.async_copy(d_hbm.at[idx_v.at[1]], rv1, s1)
        for j in range(NCH):
            b = j % 2
            cps[j].wait()
            pltpu.sync_copy(rvs[b], o_hbm.at[pl.ds(base + j * W_SC, W_SC)])
            if j + 2 < NCH:
                cps[j + 2] = pltpu.async_copy(d_hbm.at[idx_v.at[j + 2]], rvs[b], sems[b])

    return k(data, gath_idx)


def _combine_body(g_ref, a1_ref, a2_ref, y_ref):
    y_ref[...] = (g_ref[:, 0:1] * a1_ref[...].astype(jnp.float32)
                  + g_ref[:, 1:2] * a2_ref[...].astype(jnp.float32))


def _combine(gsc, gath):
    rb = 512
    return pl.pallas_call(
        _combine_body,
        grid=(S // rb,),
        in_specs=[
            pl.BlockSpec((rb, 2), lambda i: (i, 0)),
            pl.BlockSpec((rb, M), lambda i: (i, 0)),
            pl.BlockSpec((rb, M), lambda i: (i + S // rb, 0)),
        ],
        out_specs=pl.BlockSpec((rb, M), lambda i: (i, 0)),
        out_shape=jax.ShapeDtypeStruct((S, M), jnp.float32),
    )(gsc, gath, gath)


def kernel(hidden_states, wg, w1, b1, w2, b2):
    B, Sq, _ = hidden_states.shape
    x = hidden_states.reshape(S, M)
    logits = x @ wg
    idx4, gsc, laux, cnt = _gate(logits)
    scat_idx = jnp.concatenate([idx4[:, 0], idx4[:, 1]]).reshape(NW, NCH, W_SC)
    gath_idx = jnp.concatenate([idx4[:, 2], idx4[:, 3]]).reshape(NW, NCH, W_SC)
    xi = jax.lax.bitcast_convert_type(
        x.astype(jnp.bfloat16).reshape(S, MW, 2), jnp.int32)
    buf_i = _dispatch(xi, scat_idx)
    buf = jax.lax.bitcast_convert_type(buf_i, jnp.bfloat16).reshape(BUF_ROWS, M)
    out_flat = _ffn(buf, w1, b1.reshape(E, 1, F), w2, b2.reshape(E, 1, M))
    oi = jax.lax.bitcast_convert_type(
        out_flat.reshape(E * C, MW, 2), jnp.int32)
    gath_i = _gather(oi, gath_idx)
    gath = jax.lax.bitcast_convert_type(gath_i, jnp.bfloat16).reshape(ITEMS, M)
    y = _combine(gsc, gath)
    return y.reshape(B, Sq, M), laux.reshape(()), cnt.reshape(E)


# speedup vs baseline: 2.2279x; 2.2279x over previous
import functools

import jax
import jax.numpy as jnp
from jax.experimental import pallas as pl
from jax.experimental.pallas import tpu as pltpu
from jax.experimental.pallas import tpu_sc as plsc

E = 16
C = 256
S = 2048
M = 1024
F = 4096
BUF_ROWS = E * C + C
DUMP = E * C
NEG = -1e30

W_SC = 32


def _cumsum_rows(a):
    d = 1
    n = a.shape[0]
    while d < n:
        a = a + jnp.concatenate([jnp.zeros((d, a.shape[1]), a.dtype), a[: n - d]], axis=0)
        d *= 2
    return a


def _gate_body(x_ref, wg_ref, idx_ref, gsc_ref, laux_ref, cnt_ref):
    logits = jnp.dot(x_ref[...].astype(jnp.bfloat16),
                     wg_ref[...].astype(jnp.bfloat16),
                     preferred_element_type=jnp.float32)
    ids = jax.lax.broadcasted_iota(jnp.int32, (S, E), 1)
    mx1 = jnp.max(logits, axis=1, keepdims=True)
    idx1 = jnp.min(jnp.where(logits == mx1, ids, E), axis=1, keepdims=True)
    m1 = ids == idx1
    masked = jnp.where(m1, NEG, logits)
    mx2 = jnp.max(masked, axis=1, keepdims=True)
    idx2 = jnp.min(jnp.where(masked == mx2, ids, E), axis=1, keepdims=True)
    m2 = ids == idx2
    m1f = m1.astype(jnp.float32)
    m2f = m2.astype(jnp.float32)

    z = jnp.exp(logits - mx1)
    gates = z / jnp.sum(z, axis=1, keepdims=True)

    cnt1 = jnp.sum(m1f, axis=0, keepdims=True)
    loc1 = _cumsum_rows(m1f) - 1.0
    loc2 = _cumsum_rows(m2f) - 1.0 + cnt1

    me = jnp.mean(gates, axis=0, keepdims=True)
    ce = cnt1 * (1.0 / S)
    laux_ref[...] = jnp.sum(me * ce, axis=1, keepdims=True) * E
    cnt_ref[...] = (cnt1 + jnp.sum(m2f, axis=0, keepdims=True)).astype(jnp.int32)

    k1m = m1f * (loc1 < C)
    k2m = m2f * (loc2 < C)
    l1 = jnp.sum(loc1 * k1m, axis=1, keepdims=True)
    l2 = jnp.sum(loc2 * k2m, axis=1, keepdims=True)
    keep1 = jnp.sum(k1m, axis=1, keepdims=True)
    keep2 = jnp.sum(k2m, axis=1, keepdims=True)
    g1s = jnp.sum(gates * k1m, axis=1, keepdims=True)
    g2s = jnp.sum(gates * k2m, axis=1, keepdims=True)
    denom = jnp.maximum(g1s + g2s, 1e-9)
    gsc_ref[...] = jnp.concatenate(
        [g1s / denom * keep1, g2s / denom * keep2], axis=1)

    pos1 = idx1 * C + jnp.minimum(l1, C - 1).astype(jnp.int32)
    pos2 = idx2 * C + jnp.minimum(l2, C - 1).astype(jnp.int32)
    dump = jnp.int32(DUMP)
    s1 = jnp.where(keep1 > 0, pos1, dump)
    s2 = jnp.where(keep2 > 0, pos2, dump)
    idx_ref[...] = jnp.concatenate([s1, s2, pos1, pos2], axis=1)


def _gate(x, wg):
    return pl.pallas_call(
        _gate_body,
        out_shape=[
            jax.ShapeDtypeStruct((S, 4), jnp.int32),
            jax.ShapeDtypeStruct((S, 2), jnp.float32),
            jax.ShapeDtypeStruct((1, 1), jnp.float32),
            jax.ShapeDtypeStruct((1, E), jnp.int32),
        ],
    )(x, wg)


FB = 2048


def _ffn_body(buf_ref, w1_ref, b1_ref, w2_ref, b2_ref, out_ref, scr_ref):
    f = pl.program_id(1)
    nf = F // FB
    xb = buf_ref[...].astype(jnp.bfloat16)
    h = jnp.dot(xb, w1_ref[0].astype(jnp.bfloat16),
                preferred_element_type=jnp.float32)
    h = jnp.maximum(h + b1_ref[0, 0], 0.0).astype(jnp.bfloat16)
    acc = jnp.dot(h, w2_ref[0].astype(jnp.bfloat16),
                  preferred_element_type=jnp.float32)

    @pl.when(f == 0)
    def _():
        scr_ref[...] = acc

    @pl.when(f != 0)
    def _():
        scr_ref[...] += acc

    @pl.when(f == nf - 1)
    def _():
        out_ref[...] = scr_ref[...] + b2_ref[0, 0]


def _ffn(buf, w1, b1, w2, b2):
    nf = F // FB
    return pl.pallas_call(
        _ffn_body,
        grid=(E, nf),
        in_specs=[
            pl.BlockSpec((C, M), lambda e, f: (e, 0)),
            pl.BlockSpec((1, M, FB), lambda e, f: (e, 0, f)),
            pl.BlockSpec((1, 1, FB), lambda e, f: (e, 0, f)),
            pl.BlockSpec((1, FB, M), lambda e, f: (e, f, 0)),
            pl.BlockSpec((1, 1, M), lambda e, f: (e, 0, 0)),
        ],
        out_specs=pl.BlockSpec((C, M), lambda e, f: (e, 0)),
        out_shape=jax.ShapeDtypeStruct((E * C, M), jnp.float32),
        scratch_shapes=[pltpu.VMEM((C, M), jnp.float32)],
        compiler_params=pltpu.CompilerParams(
            dimension_semantics=("parallel", "arbitrary")),
    )(buf, w1, b1, w2, b2)


MW = M // 2
NW = 32
ITEMS = 2 * S
IPW = ITEMS // NW
NCH = IPW // W_SC


TPW = S // NW
NTC = TPW // W_SC


def _dispatch(x, scat_idx):
    mesh = plsc.VectorSubcoreMesh(core_axis_name="c", subcore_axis_name="s")

    @functools.partial(
        pl.kernel, mesh=mesh,
        out_type=jax.ShapeDtypeStruct((BUF_ROWS, M), jnp.float32),
        scratch_types=[
            pltpu.VMEM((2 * NTC, W_SC), jnp.int32),
            pltpu.VMEM((W_SC, M), jnp.float32),
            pltpu.VMEM((W_SC, M), jnp.float32),
            pltpu.SemaphoreType.DMA,
        ],
    )
    def k(x_hbm, i_hbm, o_hbm, idx_v, xv0, xv1, s0):
        wid = jax.lax.axis_index("s") * 2 + jax.lax.axis_index("c")
        xbase = wid * TPW
        pltpu.sync_copy(i_hbm.at[wid], idx_v)
        xvs = (xv0, xv1)
        cps = []
        for j in range(NTC):
            pltpu.sync_copy(x_hbm.at[pl.ds(xbase + j * W_SC, W_SC)], xvs[j])
            cps.append(pltpu.async_copy(xvs[j], o_hbm.at[idx_v.at[j]], s0))
            cps.append(pltpu.async_copy(xvs[j], o_hbm.at[idx_v.at[NTC + j]], s0))
        for cp in cps:
            cp.wait()

    return k(x, scat_idx)


def _gather(data, gath_idx):
    mesh = plsc.VectorSubcoreMesh(core_axis_name="c", subcore_axis_name="s")

    @functools.partial(
        pl.kernel, mesh=mesh,
        out_type=jax.ShapeDtypeStruct((ITEMS, M), jnp.float32),
        scratch_types=[
            pltpu.VMEM((NCH, W_SC), jnp.int32),
            pltpu.VMEM((W_SC, M), jnp.float32),
            pltpu.VMEM((W_SC, M), jnp.float32),
            pltpu.SemaphoreType.DMA,
            pltpu.SemaphoreType.DMA,
        ],
    )
    def k(d_hbm, i_hbm, o_hbm, idx_v, rv0, rv1, s0, s1):
        wid = jax.lax.axis_index("s") * 2 + jax.lax.axis_index("c")
        base = wid * IPW
        pltpu.sync_copy(i_hbm.at[wid], idx_v)
        rvs, sems, cps = (rv0, rv1), (s0, s1), [None] * NCH
        cps[0] = pltpu.async_copy(d_hbm.at[idx_v.at[0]], rv0, s0)
        cps[1] = pltpu.async_copy(d_hbm.at[idx_v.at[1]], rv1, s1)
        for j in range(NCH):
            b = j % 2
            cps[j].wait()
            pltpu.sync_copy(rvs[b], o_hbm.at[pl.ds(base + j * W_SC, W_SC)])
            if j + 2 < NCH:
                cps[j + 2] = pltpu.async_copy(d_hbm.at[idx_v.at[j + 2]], rvs[b], sems[b])

    return k(data, gath_idx)


def _combine_body(g_ref, a1_ref, a2_ref, y_ref):
    y_ref[...] = g_ref[:, 0:1] * a1_ref[...] + g_ref[:, 1:2] * a2_ref[...]


def _combine(gsc, gath):
    rb = 512
    return pl.pallas_call(
        _combine_body,
        grid=(S // rb,),
        in_specs=[
            pl.BlockSpec((rb, 2), lambda i: (i, 0)),
            pl.BlockSpec((rb, M), lambda i: (i, 0)),
            pl.BlockSpec((rb, M), lambda i: (i + S // rb, 0)),
        ],
        out_specs=pl.BlockSpec((rb, M), lambda i: (i, 0)),
        out_shape=jax.ShapeDtypeStruct((S, M), jnp.float32),
    )(gsc, gath, gath)


def kernel(hidden_states, wg, w1, b1, w2, b2):
    B, Sq, _ = hidden_states.shape
    x = hidden_states.reshape(S, M)
    idx4, gsc, laux, cnt = _gate(x, wg)
    scat_idx = jnp.concatenate(
        [idx4[:, 0].reshape(NW, NTC, W_SC), idx4[:, 1].reshape(NW, NTC, W_SC)],
        axis=1)
    gath_idx = jnp.concatenate([idx4[:, 2], idx4[:, 3]]).reshape(NW, NCH, W_SC)
    buf = _dispatch(x, scat_idx)
    out_flat = _ffn(buf, w1, b1.reshape(E, 1, F), w2, b2.reshape(E, 1, M))
    gath = _gather(out_flat, gath_idx)
    y = _combine(gsc, gath)
    return y.reshape(B, Sq, M), laux.reshape(()), cnt.reshape(E)


# speedup vs baseline: 2.2281x; 1.0001x over previous
import functools

import jax
import jax.numpy as jnp
from jax.experimental import pallas as pl
from jax.experimental.pallas import tpu as pltpu
from jax.experimental.pallas import tpu_sc as plsc

E = 16
C = 256
S = 2048
M = 1024
F = 4096
BUF_ROWS = E * C + C
DUMP = E * C
NEG = -1e30

W_SC = 32


def _cumsum_rows(a):
    d = 1
    n = a.shape[0]
    while d < n:
        a = a + jnp.concatenate([jnp.zeros((d, a.shape[1]), a.dtype), a[: n - d]], axis=0)
        d *= 2
    return a


def _gate_body(x_ref, wg_ref, idx_ref, gsc_ref, laux_ref, cnt_ref):
    logits = jnp.dot(x_ref[...].astype(jnp.bfloat16),
                     wg_ref[...].astype(jnp.bfloat16),
                     preferred_element_type=jnp.float32)
    ids = jax.lax.broadcasted_iota(jnp.int32, (S, E), 1)
    mx1 = jnp.max(logits, axis=1, keepdims=True)
    idx1 = jnp.min(jnp.where(logits == mx1, ids, E), axis=1, keepdims=True)
    m1 = ids == idx1
    masked = jnp.where(m1, NEG, logits)
    mx2 = jnp.max(masked, axis=1, keepdims=True)
    idx2 = jnp.min(jnp.where(masked == mx2, ids, E), axis=1, keepdims=True)
    m2 = ids == idx2
    m1f = m1.astype(jnp.float32)
    m2f = m2.astype(jnp.float32)

    z = jnp.exp(logits - mx1)
    gates = z / jnp.sum(z, axis=1, keepdims=True)

    cnt1 = jnp.sum(m1f, axis=0, keepdims=True)
    loc1 = _cumsum_rows(m1f) - 1.0
    loc2 = _cumsum_rows(m2f) - 1.0 + cnt1

    me = jnp.mean(gates, axis=0, keepdims=True)
    ce = cnt1 * (1.0 / S)
    laux_ref[...] = jnp.sum(me * ce, axis=1, keepdims=True) * E
    cnt_ref[...] = (cnt1 + jnp.sum(m2f, axis=0, keepdims=True)).astype(jnp.int32)

    k1m = m1f * (loc1 < C)
    k2m = m2f * (loc2 < C)
    l1 = jnp.sum(loc1 * k1m, axis=1, keepdims=True)
    l2 = jnp.sum(loc2 * k2m, axis=1, keepdims=True)
    keep1 = jnp.sum(k1m, axis=1, keepdims=True)
    keep2 = jnp.sum(k2m, axis=1, keepdims=True)
    g1s = jnp.sum(gates * k1m, axis=1, keepdims=True)
    g2s = jnp.sum(gates * k2m, axis=1, keepdims=True)
    denom = jnp.maximum(g1s + g2s, 1e-9)
    gsc_ref[...] = jnp.concatenate(
        [g1s / denom * keep1, g2s / denom * keep2], axis=1)

    pos1 = idx1 * C + jnp.minimum(l1, C - 1).astype(jnp.int32)
    pos2 = idx2 * C + jnp.minimum(l2, C - 1).astype(jnp.int32)
    dump = jnp.int32(DUMP)
    s1 = jnp.where(keep1 > 0, pos1, dump)
    s2 = jnp.where(keep2 > 0, pos2, dump)
    idx_ref[...] = jnp.concatenate([s1, s2, pos1, pos2], axis=1)


def _gate(x, wg):
    return pl.pallas_call(
        _gate_body,
        out_shape=[
            jax.ShapeDtypeStruct((S, 4), jnp.int32),
            jax.ShapeDtypeStruct((S, 2), jnp.float32),
            jax.ShapeDtypeStruct((1, 1), jnp.float32),
            jax.ShapeDtypeStruct((1, E), jnp.int32),
        ],
    )(x, wg)


FB = 2048


def _ffn_body(buf_ref, w1_ref, b1_ref, w2_ref, b2_ref, s12_ref, g12_ref,
              y_ref, eb_ref, acc_ref):
    e = pl.program_id(0)
    f = pl.program_id(1)
    nf = F // FB

    @pl.when(f == 0)
    def _():
        cids = jax.lax.broadcasted_iota(jnp.int32, (S, C), 1) + e * C
        ohd = ((cids == s12_ref[:, 0:1]) | (cids == s12_ref[:, 1:2]))
        ones = (jax.lax.broadcasted_iota(jnp.int32, (S, 1), 0) >= 0)
        valid = jax.lax.dot_general(
            ohd.astype(jnp.bfloat16), ones.astype(jnp.bfloat16),
            (((0,), (0,)), ((), ())), preferred_element_type=jnp.float32)
        eb_ref[...] = (buf_ref[...] * valid).astype(jnp.bfloat16)

    h = jnp.dot(eb_ref[...], w1_ref[0].astype(jnp.bfloat16),
                preferred_element_type=jnp.float32)
    h = jnp.maximum(h + b1_ref[0, 0], 0.0).astype(jnp.bfloat16)
    part = jnp.dot(h, w2_ref[0].astype(jnp.bfloat16),
                   preferred_element_type=jnp.float32)

    @pl.when(f == 0)
    def _():
        acc_ref[...] = part

    @pl.when(f != 0)
    def _():
        acc_ref[...] += part

    @pl.when(f == nf - 1)
    def _():
        oute = (acc_ref[...] + b2_ref[0, 0]).astype(jnp.bfloat16)
        cids = jax.lax.broadcasted_iota(jnp.int32, (S, C), 1) + e * C
        s1 = s12_ref[:, 0:1]
        s2 = s12_ref[:, 1:2]
        w_comb = (g12_ref[:, 0:1] * (cids == s1).astype(jnp.float32)
                  + g12_ref[:, 1:2] * (cids == s2).astype(jnp.float32))
        yp = jnp.dot(w_comb.astype(jnp.bfloat16), oute,
                     preferred_element_type=jnp.float32)

        @pl.when(e == 0)
        def _():
            y_ref[...] = yp

        @pl.when(e != 0)
        def _():
            y_ref[...] += yp


def _ffn(buf, w1, b1, w2, b2, s12, g12):
    nf = F // FB
    return pl.pallas_call(
        _ffn_body,
        grid=(E, nf),
        in_specs=[
            pl.BlockSpec((C, M), lambda e, f: (e, 0)),
            pl.BlockSpec((1, M, FB), lambda e, f: (e, 0, f)),
            pl.BlockSpec((1, 1, FB), lambda e, f: (e, 0, f)),
            pl.BlockSpec((1, FB, M), lambda e, f: (e, f, 0)),
            pl.BlockSpec((1, 1, M), lambda e, f: (e, 0, 0)),
            pl.BlockSpec((S, 2), lambda e, f: (0, 0)),
            pl.BlockSpec((S, 2), lambda e, f: (0, 0)),
        ],
        out_specs=pl.BlockSpec((S, M), lambda e, f: (0, 0)),
        out_shape=jax.ShapeDtypeStruct((S, M), jnp.float32),
        scratch_shapes=[pltpu.VMEM((C, M), jnp.bfloat16),
                        pltpu.VMEM((C, M), jnp.float32)],
        compiler_params=pltpu.CompilerParams(
            dimension_semantics=("arbitrary", "arbitrary")),
    )(buf, w1, b1, w2, b2, s12, g12)


MW = M // 2
NW = 32
ITEMS = 2 * S
IPW = ITEMS // NW
NCH = IPW // W_SC


TPW = S // NW
NTC = TPW // W_SC


def _dispatch(x, scat_idx):
    mesh = plsc.VectorSubcoreMesh(core_axis_name="c", subcore_axis_name="s")

    @functools.partial(
        pl.kernel, mesh=mesh,
        out_type=jax.ShapeDtypeStruct((BUF_ROWS, M), jnp.float32),
        scratch_types=[
            pltpu.VMEM((2 * NTC, W_SC), jnp.int32),
            pltpu.VMEM((W_SC, M), jnp.float32),
            pltpu.VMEM((W_SC, M), jnp.float32),
            pltpu.SemaphoreType.DMA,
        ],
    )
    def k(x_hbm, i_hbm, o_hbm, idx_v, xv0, xv1, s0):
        wid = jax.lax.axis_index("s") * 2 + jax.lax.axis_index("c")
        xbase = wid * TPW
        pltpu.sync_copy(i_hbm.at[wid], idx_v)
        xvs = (xv0, xv1)
        cps = []
        for j in range(NTC):
            pltpu.sync_copy(x_hbm.at[pl.ds(xbase + j * W_SC, W_SC)], xvs[j])
            cps.append(pltpu.async_copy(xvs[j], o_hbm.at[idx_v.at[j]], s0))
            cps.append(pltpu.async_copy(xvs[j], o_hbm.at[idx_v.at[NTC + j]], s0))
        for cp in cps:
            cp.wait()

    return k(x, scat_idx)


def _gather(data, gath_idx):
    mesh = plsc.VectorSubcoreMesh(core_axis_name="c", subcore_axis_name="s")

    @functools.partial(
        pl.kernel, mesh=mesh,
        out_type=jax.ShapeDtypeStruct((ITEMS, M), jnp.float32),
        scratch_types=[
            pltpu.VMEM((NCH, W_SC), jnp.int32),
            pltpu.VMEM((W_SC, M), jnp.float32),
            pltpu.VMEM((W_SC, M), jnp.float32),
            pltpu.SemaphoreType.DMA,
            pltpu.SemaphoreType.DMA,
        ],
    )
    def k(d_hbm, i_hbm, o_hbm, idx_v, rv0, rv1, s0, s1):
        wid = jax.lax.axis_index("s") * 2 + jax.lax.axis_index("c")
        base = wid * IPW
        pltpu.sync_copy(i_hbm.at[wid], idx_v)
        rvs, sems, cps = (rv0, rv1), (s0, s1), [None] * NCH
        cps[0] = pltpu.async_copy(d_hbm.at[idx_v.at[0]], rv0, s0)
        cps[1] = pltpu.async_copy(d_hbm.at[idx_v.at[1]], rv1, s1)
        for j in range(NCH):
            b = j % 2
            cps[j].wait()
            pltpu.sync_copy(rvs[b], o_hbm.at[pl.ds(base + j * W_SC, W_SC)])
            if j + 2 < NCH:
                cps[j + 2] = pltpu.async_copy(d_hbm.at[idx_v.at[j + 2]], rvs[b], sems[b])

    return k(data, gath_idx)


def _combine_body(g_ref, a1_ref, a2_ref, y_ref):
    y_ref[...] = g_ref[:, 0:1] * a1_ref[...] + g_ref[:, 1:2] * a2_ref[...]


def _combine(gsc, gath):
    rb = 512
    return pl.pallas_call(
        _combine_body,
        grid=(S // rb,),
        in_specs=[
            pl.BlockSpec((rb, 2), lambda i: (i, 0)),
            pl.BlockSpec((rb, M), lambda i: (i, 0)),
            pl.BlockSpec((rb, M), lambda i: (i + S // rb, 0)),
        ],
        out_specs=pl.BlockSpec((rb, M), lambda i: (i, 0)),
        out_shape=jax.ShapeDtypeStruct((S, M), jnp.float32),
    )(gsc, gath, gath)


def kernel(hidden_states, wg, w1, b1, w2, b2):
    B, Sq, _ = hidden_states.shape
    x = hidden_states.reshape(S, M)
    idx4, gsc, laux, cnt = _gate(x, wg)
    scat_idx = jnp.concatenate(
        [idx4[:, 0].reshape(NW, NTC, W_SC), idx4[:, 1].reshape(NW, NTC, W_SC)],
        axis=1)
    buf = _dispatch(x, scat_idx)
    y = _ffn(buf, w1, b1.reshape(E, 1, F), w2, b2.reshape(E, 1, M),
             idx4[:, 0:2], gsc)
    return y.reshape(B, Sq, M), laux.reshape(()), cnt.reshape(E)


# speedup vs baseline: 2.2418x; 1.0062x over previous
import functools

import jax
import jax.numpy as jnp
from jax.experimental import pallas as pl
from jax.experimental.pallas import tpu as pltpu
from jax.experimental.pallas import tpu_sc as plsc

E = 16
C = 256
S = 2048
M = 1024
F = 4096
BUF_ROWS = E * C + C
DUMP = E * C
NEG = -1e30

W_SC = 32


def _cumsum_rows(a):
    d = 1
    n = a.shape[0]
    while d < n:
        a = a + jnp.concatenate([jnp.zeros((d, a.shape[1]), a.dtype), a[: n - d]], axis=0)
        d *= 2
    return a


def _gate_body(x_ref, wg_ref, idx_ref, gsc_ref, laux_ref, cnt_ref, occ_ref):
    logits = jnp.dot(x_ref[...].astype(jnp.bfloat16),
                     wg_ref[...].astype(jnp.bfloat16),
                     preferred_element_type=jnp.float32)
    ids = jax.lax.broadcasted_iota(jnp.int32, (S, E), 1)
    mx1 = jnp.max(logits, axis=1, keepdims=True)
    idx1 = jnp.min(jnp.where(logits == mx1, ids, E), axis=1, keepdims=True)
    m1 = ids == idx1
    masked = jnp.where(m1, NEG, logits)
    mx2 = jnp.max(masked, axis=1, keepdims=True)
    idx2 = jnp.min(jnp.where(masked == mx2, ids, E), axis=1, keepdims=True)
    m2 = ids == idx2
    m1f = m1.astype(jnp.float32)
    m2f = m2.astype(jnp.float32)

    z = jnp.exp(logits - mx1)
    gates = z / jnp.sum(z, axis=1, keepdims=True)

    cnt1 = jnp.sum(m1f, axis=0, keepdims=True)
    loc1 = _cumsum_rows(m1f) - 1.0
    loc2 = _cumsum_rows(m2f) - 1.0 + cnt1

    me = jnp.mean(gates, axis=0, keepdims=True)
    ce = cnt1 * (1.0 / S)
    laux_ref[...] = jnp.sum(me * ce, axis=1, keepdims=True) * E
    cnt = (cnt1 + jnp.sum(m2f, axis=0, keepdims=True)).astype(jnp.int32)
    cnt_ref[...] = cnt
    occ_ref[...] = jnp.minimum(cnt, C)

    k1m = m1f * (loc1 < C)
    k2m = m2f * (loc2 < C)
    l1 = jnp.sum(loc1 * k1m, axis=1, keepdims=True)
    l2 = jnp.sum(loc2 * k2m, axis=1, keepdims=True)
    keep1 = jnp.sum(k1m, axis=1, keepdims=True)
    keep2 = jnp.sum(k2m, axis=1, keepdims=True)
    g1s = jnp.sum(gates * k1m, axis=1, keepdims=True)
    g2s = jnp.sum(gates * k2m, axis=1, keepdims=True)
    denom = jnp.maximum(g1s + g2s, 1e-9)
    gsc_ref[...] = jnp.concatenate(
        [g1s / denom * keep1, g2s / denom * keep2], axis=1)

    pos1 = idx1 * C + jnp.minimum(l1, C - 1).astype(jnp.int32)
    pos2 = idx2 * C + jnp.minimum(l2, C - 1).astype(jnp.int32)
    dump = jnp.int32(DUMP)
    s1 = jnp.where(keep1 > 0, pos1, dump)
    s2 = jnp.where(keep2 > 0, pos2, dump)
    idx_ref[...] = jnp.concatenate([s1, s2, pos1, pos2], axis=1)


def _gate(x, wg):
    return pl.pallas_call(
        _gate_body,
        out_shape=[
            jax.ShapeDtypeStruct((S, 4), jnp.int32),
            jax.ShapeDtypeStruct((S, 2), jnp.float32),
            jax.ShapeDtypeStruct((1, 1), jnp.float32),
            jax.ShapeDtypeStruct((1, E), jnp.int32),
            jax.ShapeDtypeStruct((1, E), jnp.int32),
        ],
    )(x, wg)


FB = 2048


def _ffn_body(occ_ref, buf_ref, w1_ref, b1_ref, w2_ref, b2_ref, s12_ref,
              g12_ref, y_ref, eb_ref, acc_ref):
    e = pl.program_id(0)
    f = pl.program_id(1)
    nf = F // FB

    @pl.when(f == 0)
    def _():
        rows = jax.lax.broadcasted_iota(jnp.int32, (C, 1), 0)
        valid = (rows < occ_ref[e]).astype(jnp.float32)
        eb_ref[...] = (buf_ref[...] * valid).astype(jnp.bfloat16)

    h = jnp.dot(eb_ref[...], w1_ref[0].astype(jnp.bfloat16),
                preferred_element_type=jnp.float32)
    h = jnp.maximum(h + b1_ref[0, 0], 0.0).astype(jnp.bfloat16)
    part = jnp.dot(h, w2_ref[0].astype(jnp.bfloat16),
                   preferred_element_type=jnp.float32)

    @pl.when(f == 0)
    def _():
        acc_ref[...] = part

    @pl.when(f != 0)
    def _():
        acc_ref[...] += part

    @pl.when(f == nf - 1)
    def _():
        oute = (acc_ref[...] + b2_ref[0, 0]).astype(jnp.bfloat16)
        cids = jax.lax.broadcasted_iota(jnp.int32, (S, C), 1) + e * C
        s1 = s12_ref[:, 0:1]
        s2 = s12_ref[:, 1:2]
        w_comb = (g12_ref[:, 0:1] * (cids == s1).astype(jnp.float32)
                  + g12_ref[:, 1:2] * (cids == s2).astype(jnp.float32))
        yp = jnp.dot(w_comb.astype(jnp.bfloat16), oute,
                     preferred_element_type=jnp.float32)

        @pl.when(e == 0)
        def _():
            y_ref[...] = yp

        @pl.when(e != 0)
        def _():
            y_ref[...] += yp


def _ffn(occ, buf, w1, b1, w2, b2, s12, g12):
    nf = F // FB
    return pl.pallas_call(
        _ffn_body,
        grid=(E, nf),
        in_specs=[
            pl.BlockSpec(memory_space=pltpu.SMEM),
            pl.BlockSpec((C, M), lambda e, f: (e, 0)),
            pl.BlockSpec((1, M, FB), lambda e, f: (e, 0, f)),
            pl.BlockSpec((1, 1, FB), lambda e, f: (e, 0, f)),
            pl.BlockSpec((1, FB, M), lambda e, f: (e, f, 0)),
            pl.BlockSpec((1, 1, M), lambda e, f: (e, 0, 0)),
            pl.BlockSpec((S, 2), lambda e, f: (0, 0)),
            pl.BlockSpec((S, 2), lambda e, f: (0, 0)),
        ],
        out_specs=pl.BlockSpec((S, M), lambda e, f: (0, 0)),
        out_shape=jax.ShapeDtypeStruct((S, M), jnp.float32),
        scratch_shapes=[pltpu.VMEM((C, M), jnp.bfloat16),
                        pltpu.VMEM((C, M), jnp.float32)],
        compiler_params=pltpu.CompilerParams(
            dimension_semantics=("arbitrary", "arbitrary")),
    )(occ, buf, w1, b1, w2, b2, s12, g12)


MW = M // 2
NW = 32
ITEMS = 2 * S
IPW = ITEMS // NW
NCH = IPW // W_SC


TPW = S // NW
NTC = TPW // W_SC


def _dispatch(x, scat_idx):
    mesh = plsc.VectorSubcoreMesh(core_axis_name="c", subcore_axis_name="s")

    @functools.partial(
        pl.kernel, mesh=mesh,
        out_type=jax.ShapeDtypeStruct((BUF_ROWS, M), jnp.float32),
        scratch_types=[
            pltpu.VMEM((2 * NTC, W_SC), jnp.int32),
            pltpu.VMEM((W_SC, M), jnp.float32),
            pltpu.VMEM((W_SC, M), jnp.float32),
            pltpu.SemaphoreType.DMA,
        ],
    )
    def k(x_hbm, i_hbm, o_hbm, idx_v, xv0, xv1, s0):
        wid = jax.lax.axis_index("s") * 2 + jax.lax.axis_index("c")
        xbase = wid * TPW
        pltpu.sync_copy(i_hbm.at[wid], idx_v)
        xvs = (xv0, xv1)
        cps = []
        for j in range(NTC):
            pltpu.sync_copy(x_hbm.at[pl.ds(xbase + j * W_SC, W_SC)], xvs[j])
            cps.append(pltpu.async_copy(xvs[j], o_hbm.at[idx_v.at[j]], s0))
            cps.append(pltpu.async_copy(xvs[j], o_hbm.at[idx_v.at[NTC + j]], s0))
        for cp in cps:
            cp.wait()

    return k(x, scat_idx)


def _gather(data, gath_idx):
    mesh = plsc.VectorSubcoreMesh(core_axis_name="c", subcore_axis_name="s")

    @functools.partial(
        pl.kernel, mesh=mesh,
        out_type=jax.ShapeDtypeStruct((ITEMS, M), jnp.float32),
        scratch_types=[
            pltpu.VMEM((NCH, W_SC), jnp.int32),
            pltpu.VMEM((W_SC, M), jnp.float32),
            pltpu.VMEM((W_SC, M), jnp.float32),
            pltpu.SemaphoreType.DMA,
            pltpu.SemaphoreType.DMA,
        ],
    )
    def k(d_hbm, i_hbm, o_hbm, idx_v, rv0, rv1, s0, s1):
        wid = jax.lax.axis_index("s") * 2 + jax.lax.axis_index("c")
        base = wid * IPW
        pltpu.sync_copy(i_hbm.at[wid], idx_v)
        rvs, sems, cps = (rv0, rv1), (s0, s1), [None] * NCH
        cps[0] = pltpu.async_copy(d_hbm.at[idx_v.at[0]], rv0, s0)
        cps[1] = pltpu.async_copy(d_hbm.at[idx_v.at[1]], rv1, s1)
        for j in range(NCH):
            b = j % 2
            cps[j].wait()
            pltpu.sync_copy(rvs[b], o_hbm.at[pl.ds(base + j * W_SC, W_SC)])
            if j + 2 < NCH:
                cps[j + 2] = pltpu.async_copy(d_hbm.at[idx_v.at[j + 2]], rvs[b], sems[b])

    return k(data, gath_idx)


def _combine_body(g_ref, a1_ref, a2_ref, y_ref):
    y_ref[...] = g_ref[:, 0:1] * a1_ref[...] + g_ref[:, 1:2] * a2_ref[...]


def _combine(gsc, gath):
    rb = 512
    return pl.pallas_call(
        _combine_body,
        grid=(S // rb,),
        in_specs=[
            pl.BlockSpec((rb, 2), lambda i: (i, 0)),
            pl.BlockSpec((rb, M), lambda i: (i, 0)),
            pl.BlockSpec((rb, M), lambda i: (i + S // rb, 0)),
        ],
        out_specs=pl.BlockSpec((rb, M), lambda i: (i, 0)),
        out_shape=jax.ShapeDtypeStruct((S, M), jnp.float32),
    )(gsc, gath, gath)


def kernel(hidden_states, wg, w1, b1, w2, b2):
    B, Sq, _ = hidden_states.shape
    x = hidden_states.reshape(S, M)
    idx4, gsc, laux, cnt, occ = _gate(x, wg)
    scat_idx = jnp.concatenate(
        [idx4[:, 0].reshape(NW, NTC, W_SC), idx4[:, 1].reshape(NW, NTC, W_SC)],
        axis=1)
    buf = _dispatch(x, scat_idx)
    y = _ffn(occ.reshape(E), buf, w1, b1.reshape(E, 1, F), w2,
             b2.reshape(E, 1, M), idx4[:, 0:2], gsc)
    return y.reshape(B, Sq, M), laux.reshape(()), cnt.reshape(E)


# speedup vs baseline: 2.2426x; 1.0003x over previous
import functools

import jax
import jax.numpy as jnp
from jax.experimental import pallas as pl
from jax.experimental.pallas import tpu as pltpu
from jax.experimental.pallas import tpu_sc as plsc

E = 16
C = 256
S = 2048
M = 1024
F = 4096
BUF_ROWS = E * C + C
DUMP = E * C
NEG = -1e30

W_SC = 32


def _cumsum_rows(a):
    d = 1
    n = a.shape[0]
    while d < n:
        a = a + jnp.concatenate([jnp.zeros((d, a.shape[1]), a.dtype), a[: n - d]], axis=0)
        d *= 2
    return a


def _gate_body(x_ref, wg_ref, idx_ref, gsc_ref, laux_ref, cnt_ref, occ_ref):
    logits = jnp.dot(x_ref[...].astype(jnp.bfloat16),
                     wg_ref[...].astype(jnp.bfloat16),
                     preferred_element_type=jnp.float32)
    ids = jax.lax.broadcasted_iota(jnp.int32, (S, E), 1)
    mx1 = jnp.max(logits, axis=1, keepdims=True)
    idx1 = jnp.min(jnp.where(logits == mx1, ids, E), axis=1, keepdims=True)
    m1 = ids == idx1
    masked = jnp.where(m1, NEG, logits)
    mx2 = jnp.max(masked, axis=1, keepdims=True)
    idx2 = jnp.min(jnp.where(masked == mx2, ids, E), axis=1, keepdims=True)
    m2 = ids == idx2
    m1f = m1.astype(jnp.float32)
    m2f = m2.astype(jnp.float32)

    z = jnp.exp(logits - mx1)
    gates = z / jnp.sum(z, axis=1, keepdims=True)

    cnt1 = jnp.sum(m1f, axis=0, keepdims=True)
    loc1 = _cumsum_rows(m1f) - 1.0
    loc2 = _cumsum_rows(m2f) - 1.0 + cnt1

    me = jnp.mean(gates, axis=0, keepdims=True)
    ce = cnt1 * (1.0 / S)
    laux_ref[...] = jnp.sum(me * ce, axis=1, keepdims=True) * E
    cnt = (cnt1 + jnp.sum(m2f, axis=0, keepdims=True)).astype(jnp.int32)
    cnt_ref[...] = cnt
    occ_ref[...] = jnp.minimum(cnt, C)

    k1m = m1f * (loc1 < C)
    k2m = m2f * (loc2 < C)
    l1 = jnp.sum(loc1 * k1m, axis=1, keepdims=True)
    l2 = jnp.sum(loc2 * k2m, axis=1, keepdims=True)
    keep1 = jnp.sum(k1m, axis=1, keepdims=True)
    keep2 = jnp.sum(k2m, axis=1, keepdims=True)
    g1s = jnp.sum(gates * k1m, axis=1, keepdims=True)
    g2s = jnp.sum(gates * k2m, axis=1, keepdims=True)
    denom = jnp.maximum(g1s + g2s, 1e-9)
    gsc_ref[...] = jnp.concatenate(
        [g1s / denom * keep1, g2s / denom * keep2], axis=1)

    pos1 = idx1 * C + jnp.minimum(l1, C - 1).astype(jnp.int32)
    pos2 = idx2 * C + jnp.minimum(l2, C - 1).astype(jnp.int32)
    dump = jnp.int32(DUMP)
    s1 = jnp.where(keep1 > 0, pos1, dump)
    s2 = jnp.where(keep2 > 0, pos2, dump)
    idx_ref[...] = jnp.concatenate([s1, s2, pos1, pos2], axis=1)


def _gate(x, wg):
    return pl.pallas_call(
        _gate_body,
        out_shape=[
            jax.ShapeDtypeStruct((S, 4), jnp.int32),
            jax.ShapeDtypeStruct((S, 2), jnp.float32),
            jax.ShapeDtypeStruct((1, 1), jnp.float32),
            jax.ShapeDtypeStruct((1, E), jnp.int32),
            jax.ShapeDtypeStruct((1, E), jnp.int32),
        ],
    )(x, wg)


FB = 2048


def _ffn_body(occ_ref, buf_ref, w1_ref, b1_ref, w2_ref, b2_ref, s12_ref,
              g12_ref, y_ref, eb_ref, acc_ref):
    e = pl.program_id(0)
    f = pl.program_id(1)
    nf = F // FB

    @pl.when(f == 0)
    def _():
        rows = jax.lax.broadcasted_iota(jnp.int32, (C, 1), 0)
        valid = (rows < occ_ref[e]).astype(jnp.float32)
        eb_ref[...] = (buf_ref[...] * valid).astype(jnp.bfloat16)

    h = jnp.dot(eb_ref[...], w1_ref[0].astype(jnp.bfloat16),
                preferred_element_type=jnp.float32)
    h = jnp.maximum(h + b1_ref[0, 0], 0.0).astype(jnp.bfloat16)
    part = jnp.dot(h, w2_ref[0].astype(jnp.bfloat16),
                   preferred_element_type=jnp.float32)

    @pl.when(f == 0)
    def _():
        acc_ref[...] = part

    @pl.when(f != 0)
    def _():
        acc_ref[...] += part

    @pl.when(f == nf - 1)
    def _():
        oute = (acc_ref[...] + b2_ref[0, 0]).astype(jnp.bfloat16)
        cids = jax.lax.broadcasted_iota(jnp.int32, (S, C), 1) + e * C
        s1 = s12_ref[:, 0:1]
        s2 = s12_ref[:, 1:2]
        w_comb = (g12_ref[:, 0:1] * (cids == s1).astype(jnp.float32)
                  + g12_ref[:, 1:2] * (cids == s2).astype(jnp.float32))
        yp = jnp.dot(w_comb.astype(jnp.bfloat16), oute,
                     preferred_element_type=jnp.float32)

        @pl.when(e == 0)
        def _():
            y_ref[...] = yp

        @pl.when(e != 0)
        def _():
            y_ref[...] += yp


def _ffn(occ, buf, w1, b1, w2, b2, s12, g12):
    nf = F // FB
    return pl.pallas_call(
        _ffn_body,
        grid=(E, nf),
        in_specs=[
            pl.BlockSpec(memory_space=pltpu.SMEM),
            pl.BlockSpec((C, M), lambda e, f: (e, 0)),
            pl.BlockSpec((1, M, FB), lambda e, f: (e, 0, f)),
            pl.BlockSpec((1, 1, FB), lambda e, f: (e, 0, f)),
            pl.BlockSpec((1, FB, M), lambda e, f: (e, f, 0)),
            pl.BlockSpec((1, 1, M), lambda e, f: (e, 0, 0)),
            pl.BlockSpec((S, 2), lambda e, f: (0, 0)),
            pl.BlockSpec((S, 2), lambda e, f: (0, 0)),
        ],
        out_specs=pl.BlockSpec((S, M), lambda e, f: (0, 0)),
        out_shape=jax.ShapeDtypeStruct((S, M), jnp.float32),
        scratch_shapes=[pltpu.VMEM((C, M), jnp.bfloat16),
                        pltpu.VMEM((C, M), jnp.float32)],
        compiler_params=pltpu.CompilerParams(
            dimension_semantics=("arbitrary", "arbitrary")),
    )(occ, buf, w1, b1, w2, b2, s12, g12)


MW = M // 2
NW = 32


TPW = S // NW
NTC = TPW // W_SC


def _dispatch(x, scat_idx):
    mesh = plsc.VectorSubcoreMesh(core_axis_name="c", subcore_axis_name="s")

    @functools.partial(
        pl.kernel, mesh=mesh,
        out_type=jax.ShapeDtypeStruct((BUF_ROWS, M), jnp.float32),
        scratch_types=[
            pltpu.VMEM((2 * NTC, W_SC), jnp.int32),
            pltpu.VMEM((W_SC, M), jnp.float32),
            pltpu.VMEM((W_SC, M), jnp.float32),
            pltpu.SemaphoreType.DMA,
        ],
    )
    def k(x_hbm, i_hbm, o_hbm, idx_v, xv0, xv1, s0):
        wid = jax.lax.axis_index("s") * 2 + jax.lax.axis_index("c")
        xbase = wid * TPW
        pltpu.sync_copy(i_hbm.at[wid], idx_v)
        xvs = (xv0, xv1)
        cps = []
        for j in range(NTC):
            pltpu.sync_copy(x_hbm.at[pl.ds(xbase + j * W_SC, W_SC)], xvs[j])
            cps.append(pltpu.async_copy(xvs[j], o_hbm.at[idx_v.at[j]], s0))
            cps.append(pltpu.async_copy(xvs[j], o_hbm.at[idx_v.at[NTC + j]], s0))
        for cp in cps:
            cp.wait()

    return k(x, scat_idx)


def kernel(hidden_states, wg, w1, b1, w2, b2):
    B, Sq, _ = hidden_states.shape
    x = hidden_states.reshape(S, M)
    idx4, gsc, laux, cnt, occ = _gate(x, wg)
    scat_idx = jnp.concatenate(
        [idx4[:, 0].reshape(NW, NTC, W_SC), idx4[:, 1].reshape(NW, NTC, W_SC)],
        axis=1)
    buf = _dispatch(x, scat_idx)
    y = _ffn(occ.reshape(E), buf, w1, b1.reshape(E, 1, F), w2,
             b2.reshape(E, 1, M), idx4[:, 0:2], gsc)
    return y.reshape(B, Sq, M), laux.reshape(()), cnt.reshape(E)
